# baseline (speedup 1.0000x reference)
import functools

import jax
import jax.numpy as jnp
from jax import lax
from jax.experimental import pallas as pl
from jax.experimental.pallas import tpu as pltpu
from jax.experimental.pallas import tpu_sc as plsc

_NC = 2
_NS = 16
_NW = _NC * _NS


def _make_sc_broadcast(batch: int, rows: int, dim: int):
    rows_per_w = rows // _NW
    sizes = [16, 64, 32, 64, 32, 48]
    assert sum(sizes) == rows_per_w
    chunks = []
    off = 0
    for size in sizes:
        chunks.append((off, size))
        off += size
    nchunk = len(chunks)
    mesh = plsc.VectorSubcoreMesh(core_axis_name="c", subcore_axis_name="s")

    @functools.partial(
        pl.kernel,
        out_type=jax.ShapeDtypeStruct((batch, rows, dim), jnp.float32),
        mesh=mesh,
        scratch_types=[
            pltpu.VMEM((32, dim), jnp.float32),
            pltpu.VMEM((64, dim), jnp.float32),
            pltpu.SemaphoreType.DMA,
            pltpu.SemaphoreType.DMA,
            pltpu.SemaphoreType.DMA,
            pltpu.SemaphoreType.DMA,
        ],
    )
    def sc_broadcast(table_hbm, out_hbm, buf_a, buf_b, rsem0, rsem1, wsem0, wsem1):
        wid = lax.axis_index("s") * _NC + lax.axis_index("c")
        base = wid * rows_per_w
        bufs = (buf_a, buf_b)
        rsems = (rsem0, rsem1)
        wsems = (wsem0, wsem1)

        def start_read(c):
            off, size = chunks[c]
            s = c % 2
            return pltpu.async_copy(
                table_hbm.at[pl.ds(base + off, size)],
                bufs[s].at[pl.ds(0, size)],
                rsems[s],
            )

        reads = [None] * nchunk
        writes = [None] * nchunk
        reads[0] = start_read(0)
        for c in range(nchunk):
            off, size = chunks[c]
            s = c % 2
            if c + 1 < nchunk:
                if c >= 1:
                    for h in writes[c - 1]:
                        h.wait()
                reads[c + 1] = start_read(c + 1)
            reads[c].wait()
            writes[c] = [
                pltpu.async_copy(
                    bufs[s].at[pl.ds(0, size)],
                    out_hbm.at[b, pl.ds(base + off, size)],
                    wsems[s],
                )
                for b in range(batch)
            ]
        for c in range(max(0, nchunk - 2), nchunk):
            for h in writes[c]:
                h.wait()

    return sc_broadcast


def kernel(x, pos_embedding_weight):
    batch, seq_len = x.shape
    rows, dim = pos_embedding_weight.shape
    fn = _make_sc_broadcast(batch, rows, dim)
    return fn(pos_embedding_weight)


# speedup vs baseline: 1.0108x; 1.0108x over previous
import functools

import jax
import jax.numpy as jnp
from jax import lax
from jax.experimental import pallas as pl
from jax.experimental.pallas import tpu as pltpu
from jax.experimental.pallas import tpu_sc as plsc

_NC = 2
_NS = 16
_NW = _NC * _NS


def _make_sc_broadcast(batch: int, rows: int, dim: int):
    rows_per_w = rows // _NW
    sizes = [8, 64, 32, 64, 32, 56]
    assert sum(sizes) == rows_per_w
    chunks = []
    off = 0
    for size in sizes:
        chunks.append((off, size))
        off += size
    nchunk = len(chunks)
    mesh = plsc.VectorSubcoreMesh(core_axis_name="c", subcore_axis_name="s")

    @functools.partial(
        pl.kernel,
        out_type=jax.ShapeDtypeStruct((batch, rows, dim), jnp.float32),
        mesh=mesh,
        scratch_types=[
            pltpu.VMEM((32, dim), jnp.float32),
            pltpu.VMEM((64, dim), jnp.float32),
            pltpu.SemaphoreType.DMA,
            pltpu.SemaphoreType.DMA,
            pltpu.SemaphoreType.DMA,
            pltpu.SemaphoreType.DMA,
        ],
    )
    def sc_broadcast(table_hbm, out_hbm, buf_a, buf_b, rsem0, rsem1, wsem0, wsem1):
        wid = lax.axis_index("s") * _NC + lax.axis_index("c")
        base = wid * rows_per_w
        bufs = (buf_a, buf_b)
        rsems = (rsem0, rsem1)
        wsems = (wsem0, wsem1)

        def start_read(c):
            off, size = chunks[c]
            s = c % 2
            return pltpu.async_copy(
                table_hbm.at[pl.ds(base + off, size)],
                bufs[s].at[pl.ds(0, size)],
                rsems[s],
            )

        reads = [None] * nchunk
        writes = [None] * nchunk
        reads[0] = start_read(0)
        for c in range(nchunk):
            off, size = chunks[c]
            s = c % 2
            if c + 1 < nchunk:
                if c >= 1:
                    for h in writes[c - 1]:
                        h.wait()
                reads[c + 1] = start_read(c + 1)
            reads[c].wait()
            writes[c] = [
                pltpu.async_copy(
                    bufs[s].at[pl.ds(0, size)],
                    out_hbm.at[b, pl.ds(base + off, size)],
                    wsems[s],
                )
                for b in range(batch)
            ]
        for c in range(max(0, nchunk - 2), nchunk):
            for h in writes[c]:
                h.wait()

    return sc_broadcast


def kernel(x, pos_embedding_weight):
    batch, seq_len = x.shape
    rows, dim = pos_embedding_weight.shape
    fn = _make_sc_broadcast(batch, rows, dim)
    return fn(pos_embedding_weight)
